# Initial kernel scaffold; baseline (speedup 1.0000x reference)
import jax
import jax.numpy as jnp
from jax.experimental import pallas as pl


def kernel(inp, gate_idx, gate_score, W, b):
    raise NotImplementedError("write your pallas kernel here")


# speedup vs baseline: 75.8690x; 75.8690x over previous
import jax
import jax.numpy as jnp
from jax.experimental import pallas as pl


def _moe_dense_body(idx_ref, score_ref, x_ref, w_ref, b_ref, o_ref):
    e = pl.program_id(0)
    match = (idx_ref[...] == e).astype(jnp.float32)
    coeff = jnp.sum(score_ref[...] * match, axis=1, keepdims=True)
    y = jax.lax.dot_general(
        x_ref[...], w_ref[0],
        dimension_numbers=(((1,), (1,)), ((), ())),
        preferred_element_type=jnp.float32,
    )
    contrib = coeff * (y + b_ref[0])

    @pl.when(e == 0)
    def _():
        o_ref[...] = contrib

    @pl.when(e != 0)
    def _():
        o_ref[...] += contrib


def kernel(inp, gate_idx, gate_score, W, b):
    T, D = inp.shape
    E = W.shape[0]
    K = gate_idx.shape[1]

    return pl.pallas_call(
        _moe_dense_body,
        grid=(E,),
        in_specs=[
            pl.BlockSpec((T, K), lambda e: (0, 0)),
            pl.BlockSpec((T, K), lambda e: (0, 0)),
            pl.BlockSpec((T, D), lambda e: (0, 0)),
            pl.BlockSpec((1, D, D), lambda e: (e, 0, 0)),
            pl.BlockSpec((1, 1, D), lambda e: (e, 0, 0)),
        ],
        out_specs=pl.BlockSpec((T, D), lambda e: (0, 0)),
        out_shape=jax.ShapeDtypeStruct((T, D), jnp.float32),
    )(gate_idx, gate_score, inp, W, b.reshape(E, 1, D))
